# baseline (speedup 1.0000x reference)
import functools

import jax
import jax.numpy as jnp
from jax.experimental import pallas as pl
from jax.experimental.pallas import tpu as pltpu

_FP8 = jnp.float8_e4m3fn
_SA = 440.0


def _first_layer_body(h_ref, w_ref, b_ref, adj_ref, out_ref, adjq_ref,
                      rowsum_ref, rowsumc_ref, support_ref):
    @pl.when(pl.program_id(0) == 0)
    def _():
        support_ref[...] = jnp.dot(h_ref[...], w_ref[...],
                                   preferred_element_type=jnp.float32)

    a = adj_ref[...]
    aq = ((a - 0.5) * _SA).astype(_FP8)
    adjq_ref[...] = aq
    rowsum_ref[...] = jnp.sum(a, axis=1, keepdims=True)
    rowsumc_ref[...] = jnp.sum(aq.astype(jnp.float32), axis=1, keepdims=True)
    acc = jnp.dot(a, support_ref[...], preferred_element_type=jnp.float32)
    out_ref[...] = jnp.maximum(acc + b_ref[...], 0.0)


def _layer_body(h_ref, w_ref, b_ref, adj_ref, rowsum_ref, rowsumc_ref,
                out_ref, support_ref, siga_ref, dmu_ref, mu_ref, base_ref,
                *, last):
    @pl.when(pl.program_id(0) == 0)
    def _():
        s = jnp.dot(h_ref[...], w_ref[...], preferred_element_type=jnp.float32)
        mu = jnp.mean(s, axis=0, keepdims=True)
        s_c = s - mu
        sig = jnp.max(jnp.abs(s_c), axis=0, keepdims=True) * (1.0 / 240.0)
        sig = jnp.maximum(sig, 1e-30)
        s_scaled = s_c * (1.0 / sig)
        sq = s_scaled.astype(_FP8)
        support_ref[...] = sq
        siga_ref[...] = sig * (1.0 / _SA)
        dmu_ref[...] = jnp.mean(s_scaled - sq.astype(jnp.float32), axis=0,
                                keepdims=True)
        mu_ref[...] = mu
        base_ref[...] = 0.5 * jnp.sum(s_c, axis=0, keepdims=True) + b_ref[...]

    acc = jnp.dot(adj_ref[...], support_ref[...],
                  preferred_element_type=jnp.float32)
    acc = acc + rowsumc_ref[...] * dmu_ref[...]
    logits = (acc * siga_ref[...] + rowsum_ref[...] * mu_ref[...]
              + base_ref[...])
    if last:
        m = jnp.max(logits, axis=1, keepdims=True)
        lse = jnp.log(jnp.sum(jnp.exp(logits - m), axis=1, keepdims=True)) + m
        out_ref[...] = logits - lse
    else:
        out_ref[...] = jnp.maximum(logits, 0.0)


def _first_layer(x, adj, W, b, *, block):
    n, nin = x.shape
    nout = W.shape[1]
    grid = n // block
    return pl.pallas_call(
        _first_layer_body,
        grid=(grid,),
        in_specs=[
            pl.BlockSpec((n, nin), lambda i: (0, 0)),
            pl.BlockSpec((nin, nout), lambda i: (0, 0)),
            pl.BlockSpec((1, nout), lambda i: (0, 0)),
            pl.BlockSpec((block, n), lambda i: (i, 0)),
        ],
        out_specs=[
            pl.BlockSpec((block, nout), lambda i: (i, 0)),
            pl.BlockSpec((block, n), lambda i: (i, 0)),
            pl.BlockSpec((block, 1), lambda i: (i, 0)),
            pl.BlockSpec((block, 1), lambda i: (i, 0)),
        ],
        out_shape=[
            jax.ShapeDtypeStruct((n, nout), jnp.float32),
            jax.ShapeDtypeStruct((n, n), _FP8),
            jax.ShapeDtypeStruct((n, 1), jnp.float32),
            jax.ShapeDtypeStruct((n, 1), jnp.float32),
        ],
        scratch_shapes=[pltpu.VMEM((n, nout), jnp.float32)],
        compiler_params=pltpu.CompilerParams(
            dimension_semantics=("arbitrary",),
        ),
    )(x, W, b.reshape(1, nout), adj)


def _layer(h, adj_q, rowsum, rowsumc, W, b, *, last, block):
    n, nin = h.shape
    nout = W.shape[1]
    grid = n // block
    body = functools.partial(_layer_body, last=last)
    return pl.pallas_call(
        body,
        grid=(grid,),
        in_specs=[
            pl.BlockSpec((n, nin), lambda i: (0, 0)),
            pl.BlockSpec((nin, nout), lambda i: (0, 0)),
            pl.BlockSpec((1, nout), lambda i: (0, 0)),
            pl.BlockSpec((block, n), lambda i: (i, 0)),
            pl.BlockSpec((block, 1), lambda i: (i, 0)),
            pl.BlockSpec((block, 1), lambda i: (i, 0)),
        ],
        out_specs=pl.BlockSpec((block, nout), lambda i: (i, 0)),
        out_shape=jax.ShapeDtypeStruct((n, nout), jnp.float32),
        scratch_shapes=[
            pltpu.VMEM((n, nout), _FP8),
            pltpu.VMEM((1, nout), jnp.float32),
            pltpu.VMEM((1, nout), jnp.float32),
            pltpu.VMEM((1, nout), jnp.float32),
            pltpu.VMEM((1, nout), jnp.float32),
        ],
        compiler_params=pltpu.CompilerParams(
            dimension_semantics=("arbitrary",),
        ),
    )(h, W, b.reshape(1, nout), adj_q, rowsum, rowsumc)


def kernel(x, adj, W1, b1, W2, b2, W3, b3, W4, b4, W5, b5, W6, b6):
    n = adj.shape[0]
    block1 = 400 if n % 400 == 0 else n
    block = 1000 if n % 1000 == 0 else n
    h, adj_q, rowsum, rowsumc = _first_layer(x, adj, W1, b1, block=block1)
    for W, b in ((W2, b2), (W3, b3), (W4, b4), (W5, b5)):
        h = _layer(h, adj_q, rowsum, rowsumc, W, b, last=False, block=block)
    return _layer(h, adj_q, rowsum, rowsumc, W6, b6, last=True, block=block)


# speedup vs baseline: 1.0291x; 1.0291x over previous
import functools

import jax
import jax.numpy as jnp
from jax.experimental import pallas as pl
from jax.experimental.pallas import tpu as pltpu

_FP8 = jnp.float8_e4m3fn
_SA = 440.0


def _first_layer_body(h_ref, w_ref, b_ref, adj_ref, out_ref, adjq_ref,
                      rowsum_ref, support_ref):
    @pl.when(pl.program_id(0) == 0)
    def _():
        support_ref[...] = jnp.dot(h_ref[...], w_ref[...],
                                   preferred_element_type=jnp.float32)

    a = adj_ref[...]
    adjq_ref[...] = ((a - 0.5) * _SA).astype(_FP8)
    rowsum_ref[...] = jnp.sum(a, axis=1, keepdims=True)
    acc = jnp.dot(a, support_ref[...], preferred_element_type=jnp.float32)
    out_ref[...] = jnp.maximum(acc + b_ref[...], 0.0)


def _quant_body(h_ref, w_ref, sq_ref, siga_ref, mu_ref):
    s = jnp.dot(h_ref[...], w_ref[...], preferred_element_type=jnp.float32)
    mu = jnp.mean(s, axis=0, keepdims=True)
    s_c = s - mu
    sig = jnp.max(jnp.abs(s_c), axis=0, keepdims=True) * (1.0 / 240.0)
    sig = jnp.maximum(sig, 1e-30)
    sq_ref[...] = (s_c * (1.0 / sig)).astype(_FP8)
    siga_ref[...] = sig * (1.0 / _SA)
    mu_ref[...] = mu


def _stream_body(sq_ref, siga_ref, mu_ref, b_ref, adj_ref, rowsum_ref,
                 out_ref, *, last):
    acc = jnp.dot(adj_ref[...], sq_ref[...],
                  preferred_element_type=jnp.float32)
    logits = (acc * siga_ref[...] + rowsum_ref[...] * mu_ref[...]
              + b_ref[...])
    if last:
        m = jnp.max(logits, axis=1, keepdims=True)
        lse = jnp.log(jnp.sum(jnp.exp(logits - m), axis=1, keepdims=True)) + m
        out_ref[...] = logits - lse
    else:
        out_ref[...] = jnp.maximum(logits, 0.0)


def _first_layer(x, adj, W, b, *, block):
    n, nin = x.shape
    nout = W.shape[1]
    grid = n // block
    return pl.pallas_call(
        _first_layer_body,
        grid=(grid,),
        in_specs=[
            pl.BlockSpec((n, nin), lambda i: (0, 0)),
            pl.BlockSpec((nin, nout), lambda i: (0, 0)),
            pl.BlockSpec((1, nout), lambda i: (0, 0)),
            pl.BlockSpec((block, n), lambda i: (i, 0)),
        ],
        out_specs=[
            pl.BlockSpec((block, nout), lambda i: (i, 0)),
            pl.BlockSpec((block, n), lambda i: (i, 0)),
            pl.BlockSpec((block, 1), lambda i: (i, 0)),
        ],
        out_shape=[
            jax.ShapeDtypeStruct((n, nout), jnp.float32),
            jax.ShapeDtypeStruct((n, n), _FP8),
            jax.ShapeDtypeStruct((n, 1), jnp.float32),
        ],
        scratch_shapes=[pltpu.VMEM((n, nout), jnp.float32)],
        compiler_params=pltpu.CompilerParams(
            dimension_semantics=("arbitrary",),
        ),
    )(x, W, b.reshape(1, nout), adj)


def _quant_support(h, W):
    n, nin = h.shape
    nout = W.shape[1]
    return pl.pallas_call(
        _quant_body,
        out_shape=[
            jax.ShapeDtypeStruct((n, nout), _FP8),
            jax.ShapeDtypeStruct((1, nout), jnp.float32),
            jax.ShapeDtypeStruct((1, nout), jnp.float32),
        ],
    )(h, W)


def _stream_layer(sq, siga, mu, adj_q, rowsum, b, *, last, block):
    n, nout = sq.shape
    grid = n // block
    body = functools.partial(_stream_body, last=last)
    return pl.pallas_call(
        body,
        grid=(grid,),
        in_specs=[
            pl.BlockSpec((n, nout), lambda i: (0, 0)),
            pl.BlockSpec((1, nout), lambda i: (0, 0)),
            pl.BlockSpec((1, nout), lambda i: (0, 0)),
            pl.BlockSpec((1, nout), lambda i: (0, 0)),
            pl.BlockSpec((block, n), lambda i: (i, 0)),
            pl.BlockSpec((block, 1), lambda i: (i, 0)),
        ],
        out_specs=pl.BlockSpec((block, nout), lambda i: (i, 0)),
        out_shape=jax.ShapeDtypeStruct((n, nout), jnp.float32),
        compiler_params=pltpu.CompilerParams(
            dimension_semantics=("arbitrary",),
        ),
    )(sq, siga, mu, b.reshape(1, nout), adj_q, rowsum)


def kernel(x, adj, W1, b1, W2, b2, W3, b3, W4, b4, W5, b5, W6, b6):
    n = adj.shape[0]
    block1 = 400 if n % 400 == 0 else n
    block = 1000 if n % 1000 == 0 else n
    h, adj_q, rowsum = _first_layer(x, adj, W1, b1, block=block1)
    for W, b, last in ((W2, b2, False), (W3, b3, False), (W4, b4, False),
                       (W5, b5, False), (W6, b6, True)):
        sq, siga, mu = _quant_support(h, W)
        h = _stream_layer(sq, siga, mu, adj_q, rowsum, b, last=last,
                          block=block)
    return h


# speedup vs baseline: 1.0454x; 1.0157x over previous
import functools

import jax
import jax.numpy as jnp
from jax.experimental import pallas as pl
from jax.experimental.pallas import tpu as pltpu

_FP8 = jnp.float8_e4m3fn
_SA = 440.0


def _first_layer_body(h_ref, w_ref, b_ref, adj_ref, out_ref, adjq_ref,
                      rowsum_ref, support_ref):
    @pl.when(pl.program_id(0) == 0)
    def _():
        support_ref[...] = jnp.dot(h_ref[...], w_ref[...],
                                   preferred_element_type=jnp.float32)

    a = adj_ref[...]
    adjq_ref[...] = ((a - 0.5) * _SA).astype(_FP8)
    rowsum_ref[...] = jnp.sum(a, axis=1, keepdims=True)
    acc = jnp.dot(a, support_ref[...], preferred_element_type=jnp.float32)
    out_ref[...] = jnp.maximum(acc + b_ref[...], 0.0).astype(jnp.bfloat16)


def _quant_body(h_ref, w_ref, sq_ref, siga_ref, mu_ref):
    s = jnp.dot(h_ref[...].astype(jnp.float32), w_ref[...],
                preferred_element_type=jnp.float32)
    mu = jnp.mean(s, axis=0, keepdims=True)
    s_c = s - mu
    sig = jnp.max(jnp.abs(s_c), axis=0, keepdims=True) * (1.0 / 240.0)
    sig = jnp.maximum(sig, 1e-30)
    sq_ref[...] = (s_c * (1.0 / sig)).astype(_FP8)
    siga_ref[...] = sig * (1.0 / _SA)
    mu_ref[...] = mu


def _stream_body(sq_ref, siga_ref, mu_ref, b_ref, adj_ref, rowsum_ref,
                 out_ref, *, last):
    acc = jnp.dot(adj_ref[...], sq_ref[...],
                  preferred_element_type=jnp.float32)
    logits = (acc * siga_ref[...] + rowsum_ref[...] * mu_ref[...]
              + b_ref[...])
    if last:
        m = jnp.max(logits, axis=1, keepdims=True)
        lse = jnp.log(jnp.sum(jnp.exp(logits - m), axis=1, keepdims=True)) + m
        out_ref[...] = logits - lse
    else:
        out_ref[...] = jnp.maximum(logits, 0.0).astype(jnp.bfloat16)


def _first_layer(x, adj, W, b, *, block):
    n, nin = x.shape
    nout = W.shape[1]
    grid = n // block
    return pl.pallas_call(
        _first_layer_body,
        grid=(grid,),
        in_specs=[
            pl.BlockSpec((n, nin), lambda i: (0, 0)),
            pl.BlockSpec((nin, nout), lambda i: (0, 0)),
            pl.BlockSpec((1, nout), lambda i: (0, 0)),
            pl.BlockSpec((block, n), lambda i: (i, 0)),
        ],
        out_specs=[
            pl.BlockSpec((block, nout), lambda i: (i, 0)),
            pl.BlockSpec((block, n), lambda i: (i, 0)),
            pl.BlockSpec((block, 1), lambda i: (i, 0)),
        ],
        out_shape=[
            jax.ShapeDtypeStruct((n, nout), jnp.bfloat16),
            jax.ShapeDtypeStruct((n, n), _FP8),
            jax.ShapeDtypeStruct((n, 1), jnp.float32),
        ],
        scratch_shapes=[pltpu.VMEM((n, nout), jnp.float32)],
        compiler_params=pltpu.CompilerParams(
            dimension_semantics=("arbitrary",),
        ),
    )(x, W, b.reshape(1, nout), adj)


def _quant_support(h, W):
    n, nin = h.shape
    nout = W.shape[1]
    return pl.pallas_call(
        _quant_body,
        out_shape=[
            jax.ShapeDtypeStruct((n, nout), _FP8),
            jax.ShapeDtypeStruct((1, nout), jnp.float32),
            jax.ShapeDtypeStruct((1, nout), jnp.float32),
        ],
    )(h, W)


def _stream_layer(sq, siga, mu, adj_q, rowsum, b, *, last, block):
    n, nout = sq.shape
    grid = n // block
    body = functools.partial(_stream_body, last=last)
    return pl.pallas_call(
        body,
        grid=(grid,),
        in_specs=[
            pl.BlockSpec((n, nout), lambda i: (0, 0)),
            pl.BlockSpec((1, nout), lambda i: (0, 0)),
            pl.BlockSpec((1, nout), lambda i: (0, 0)),
            pl.BlockSpec((1, nout), lambda i: (0, 0)),
            pl.BlockSpec((block, n), lambda i: (i, 0)),
            pl.BlockSpec((block, 1), lambda i: (i, 0)),
        ],
        out_specs=pl.BlockSpec((block, nout), lambda i: (i, 0)),
        out_shape=jax.ShapeDtypeStruct(
            (n, nout), jnp.float32 if last else jnp.bfloat16),
        compiler_params=pltpu.CompilerParams(
            dimension_semantics=("parallel",),
        ),
    )(sq, siga, mu, b.reshape(1, nout), adj_q, rowsum)


def kernel(x, adj, W1, b1, W2, b2, W3, b3, W4, b4, W5, b5, W6, b6):
    n = adj.shape[0]
    block1 = 400 if n % 400 == 0 else n
    block = 1000 if n % 1000 == 0 else n
    h, adj_q, rowsum = _first_layer(x, adj, W1, b1, block=block1)
    for W, b, last in ((W2, b2, False), (W3, b3, False), (W4, b4, False),
                       (W5, b5, False), (W6, b6, True)):
        sq, siga, mu = _quant_support(h, W)
        h = _stream_layer(sq, siga, mu, adj_q, rowsum, b, last=last,
                          block=block)
    return h


# speedup vs baseline: 1.0810x; 1.0341x over previous
import functools

import jax
import jax.numpy as jnp
from jax.experimental import pallas as pl
from jax.experimental.pallas import tpu as pltpu

_FP8 = jnp.float8_e4m3fn
_SA = 440.0


def _quantize_next(s, c_ref, isig_ref, sq_ref, siga_ref, mu_ref):
    @pl.when(pl.program_id(0) == 0)
    def _():
        mx = jnp.max(s, axis=0, keepdims=True)
        mn = jnp.min(s, axis=0, keepdims=True)
        c = (mx + mn) * 0.5
        sig = jnp.maximum((mx - mn) * (1.0 / 240.0), 1e-30)
        c_ref[...] = c
        isig_ref[...] = 1.0 / sig
        siga_ref[...] = sig * (1.0 / _SA)
        mu_ref[...] = c

    s_scaled = (s - c_ref[...]) * isig_ref[...]
    sq_ref[...] = jnp.clip(s_scaled, -440.0, 440.0).astype(_FP8)


def _accum_colsum_and_emit(h, wn_ref, ch_ref, csn_ref, *, grid):
    i = pl.program_id(0)
    hs = jnp.sum(h, axis=0, keepdims=True)

    @pl.when(i == 0)
    def _():
        ch_ref[...] = hs

    @pl.when(i != 0)
    def _():
        ch_ref[...] = ch_ref[...] + hs

    @pl.when(i == grid - 1)
    def _():
        csn_ref[...] = jnp.dot(ch_ref[...], wn_ref[...],
                               preferred_element_type=jnp.float32)


def _first_layer_body(x_ref, w1_ref, b1_ref, w2_ref, adj_ref,
                      adjq_ref, rowsum_ref, sq_ref, siga_ref, mu_ref,
                      cs_ref, support_ref, c_ref, isig_ref, ch_ref, *, grid):
    @pl.when(pl.program_id(0) == 0)
    def _():
        support_ref[...] = jnp.dot(x_ref[...], w1_ref[...],
                                   preferred_element_type=jnp.float32)

    a = adj_ref[...]
    adjq_ref[...] = ((a - 0.5) * _SA).astype(_FP8)
    rowsum_ref[...] = jnp.sum(a, axis=1, keepdims=True)
    acc = jnp.dot(a, support_ref[...], preferred_element_type=jnp.float32)
    h = jnp.maximum(acc + b1_ref[...], 0.0)
    s = jnp.dot(h, w2_ref[...], preferred_element_type=jnp.float32)
    _quantize_next(s, c_ref, isig_ref, sq_ref, siga_ref, mu_ref)
    _accum_colsum_and_emit(h, w2_ref, ch_ref, cs_ref, grid=grid)


def _epilogue_logits(acc, siga_ref, mu_ref, cs_ref, b_ref, rowsum_ref, *, n):
    base = 0.5 * cs_ref[...] - (0.5 * n) * mu_ref[...] + b_ref[...]
    return acc * siga_ref[...] + rowsum_ref[...] * mu_ref[...] + base


def _stream_body(sq_ref, siga_ref, mu_ref, cs_ref, b_ref, wn_ref, adj_ref,
                 rowsum_ref, sqn_ref, sigan_ref, mun_ref, csn_ref,
                 c_ref, isig_ref, ch_ref, *, n, grid):
    acc = jnp.dot(adj_ref[...], sq_ref[...],
                  preferred_element_type=jnp.float32)
    logits = _epilogue_logits(acc, siga_ref, mu_ref, cs_ref, b_ref,
                              rowsum_ref, n=n)
    h = jnp.maximum(logits, 0.0)
    s = jnp.dot(h, wn_ref[...], preferred_element_type=jnp.float32)
    _quantize_next(s, c_ref, isig_ref, sqn_ref, sigan_ref, mun_ref)
    _accum_colsum_and_emit(h, wn_ref, ch_ref, csn_ref, grid=grid)


def _last_body(sq_ref, siga_ref, mu_ref, cs_ref, b_ref, adj_ref, rowsum_ref,
               out_ref, *, n):
    acc = jnp.dot(adj_ref[...], sq_ref[...],
                  preferred_element_type=jnp.float32)
    logits = _epilogue_logits(acc, siga_ref, mu_ref, cs_ref, b_ref,
                              rowsum_ref, n=n)
    m = jnp.max(logits, axis=1, keepdims=True)
    lse = jnp.log(jnp.sum(jnp.exp(logits - m), axis=1, keepdims=True)) + m
    out_ref[...] = logits - lse


def _first_layer(x, adj, W1, b1, W2, *, block):
    n, nin = x.shape
    nout = W1.shape[1]
    nnext = W2.shape[1]
    grid = n // block
    body = functools.partial(_first_layer_body, grid=grid)
    return pl.pallas_call(
        body,
        grid=(grid,),
        in_specs=[
            pl.BlockSpec((n, nin), lambda i: (0, 0)),
            pl.BlockSpec((nin, nout), lambda i: (0, 0)),
            pl.BlockSpec((1, nout), lambda i: (0, 0)),
            pl.BlockSpec((nout, nnext), lambda i: (0, 0)),
            pl.BlockSpec((block, n), lambda i: (i, 0)),
        ],
        out_specs=[
            pl.BlockSpec((block, n), lambda i: (i, 0)),
            pl.BlockSpec((block, 1), lambda i: (i, 0)),
            pl.BlockSpec((block, nnext), lambda i: (i, 0)),
            pl.BlockSpec((1, nnext), lambda i: (0, 0)),
            pl.BlockSpec((1, nnext), lambda i: (0, 0)),
            pl.BlockSpec((1, nnext), lambda i: (0, 0)),
        ],
        out_shape=[
            jax.ShapeDtypeStruct((n, n), _FP8),
            jax.ShapeDtypeStruct((n, 1), jnp.float32),
            jax.ShapeDtypeStruct((n, nnext), _FP8),
            jax.ShapeDtypeStruct((1, nnext), jnp.float32),
            jax.ShapeDtypeStruct((1, nnext), jnp.float32),
            jax.ShapeDtypeStruct((1, nnext), jnp.float32),
        ],
        scratch_shapes=[
            pltpu.VMEM((n, nout), jnp.float32),
            pltpu.VMEM((1, nnext), jnp.float32),
            pltpu.VMEM((1, nnext), jnp.float32),
            pltpu.VMEM((1, nout), jnp.float32),
        ],
        compiler_params=pltpu.CompilerParams(
            dimension_semantics=("arbitrary",),
        ),
    )(x, W1, b1.reshape(1, nout), W2, adj)


def _stream_layer(sq, siga, mu, cs, adj_q, rowsum, b, Wn, *, block):
    n, nout = sq.shape
    nnext = Wn.shape[1]
    grid = n // block
    body = functools.partial(_stream_body, n=n, grid=grid)
    return pl.pallas_call(
        body,
        grid=(grid,),
        in_specs=[
            pl.BlockSpec((n, nout), lambda i: (0, 0)),
            pl.BlockSpec((1, nout), lambda i: (0, 0)),
            pl.BlockSpec((1, nout), lambda i: (0, 0)),
            pl.BlockSpec((1, nout), lambda i: (0, 0)),
            pl.BlockSpec((1, nout), lambda i: (0, 0)),
            pl.BlockSpec((nout, nnext), lambda i: (0, 0)),
            pl.BlockSpec((block, n), lambda i: (i, 0)),
            pl.BlockSpec((block, 1), lambda i: (i, 0)),
        ],
        out_specs=[
            pl.BlockSpec((block, nnext), lambda i: (i, 0)),
            pl.BlockSpec((1, nnext), lambda i: (0, 0)),
            pl.BlockSpec((1, nnext), lambda i: (0, 0)),
            pl.BlockSpec((1, nnext), lambda i: (0, 0)),
        ],
        out_shape=[
            jax.ShapeDtypeStruct((n, nnext), _FP8),
            jax.ShapeDtypeStruct((1, nnext), jnp.float32),
            jax.ShapeDtypeStruct((1, nnext), jnp.float32),
            jax.ShapeDtypeStruct((1, nnext), jnp.float32),
        ],
        scratch_shapes=[
            pltpu.VMEM((1, nnext), jnp.float32),
            pltpu.VMEM((1, nnext), jnp.float32),
            pltpu.VMEM((1, nout), jnp.float32),
        ],
        compiler_params=pltpu.CompilerParams(
            dimension_semantics=("arbitrary",),
        ),
    )(sq, siga, mu, cs, b.reshape(1, nout), Wn, adj_q, rowsum)


def _last_layer(sq, siga, mu, cs, adj_q, rowsum, b, *, block):
    n, nout = sq.shape
    grid = n // block
    body = functools.partial(_last_body, n=n)
    return pl.pallas_call(
        body,
        grid=(grid,),
        in_specs=[
            pl.BlockSpec((n, nout), lambda i: (0, 0)),
            pl.BlockSpec((1, nout), lambda i: (0, 0)),
            pl.BlockSpec((1, nout), lambda i: (0, 0)),
            pl.BlockSpec((1, nout), lambda i: (0, 0)),
            pl.BlockSpec((1, nout), lambda i: (0, 0)),
            pl.BlockSpec((block, n), lambda i: (i, 0)),
            pl.BlockSpec((block, 1), lambda i: (i, 0)),
        ],
        out_specs=pl.BlockSpec((block, nout), lambda i: (i, 0)),
        out_shape=jax.ShapeDtypeStruct((n, nout), jnp.float32),
        compiler_params=pltpu.CompilerParams(
            dimension_semantics=("parallel",),
        ),
    )(sq, siga, mu, cs, b.reshape(1, nout), adj_q, rowsum)


def kernel(x, adj, W1, b1, W2, b2, W3, b3, W4, b4, W5, b5, W6, b6):
    n = adj.shape[0]
    block1 = 400 if n % 400 == 0 else n
    block = 1000 if n % 1000 == 0 else n
    adj_q, rowsum, sq, siga, mu, cs = _first_layer(x, adj, W1, b1, W2,
                                                   block=block1)
    for b, Wn in ((b2, W3), (b3, W4), (b4, W5), (b5, W6)):
        sq, siga, mu, cs = _stream_layer(sq, siga, mu, cs, adj_q, rowsum,
                                         b, Wn, block=block)
    return _last_layer(sq, siga, mu, cs, adj_q, rowsum, b6, block=block)
